# baseline (speedup 1.0000x reference)
import functools

import jax
import jax.numpy as jnp
from jax.experimental import pallas as pl
from jax.experimental.pallas import tpu as pltpu

N, T, F_IN, D = 2000, 32, 64, 128
M_HYPER, E_EXP, H_GAT, H_ATTN = 32, 4, 4, 2
O_GAT = D // H_GAT
DH_ATTN = D // H_ATTN

BN_MAGE = 200
BN_GAT = 400
LN_EPS = 1e-5


def _ln_rows(x, g, b):
    mu = x.mean(-1, keepdims=True)
    var = ((x - mu) ** 2).mean(-1, keepdims=True)
    return (x - mu) / jnp.sqrt(var + LN_EPS) * g + b


def _mm(a, b):
    return jax.lax.dot_general(a, b, (((1,), (0,)), ((), ())),
                               preferred_element_type=jnp.float32)


def _mage_kernel(xT_ref, WinT_ref, bin_ref, Wcf_ref, bcf_ref, Wcb_ref, bcb_ref,
                 WgfT_ref, bgf_ref, WgbT_ref, g_gru_ref, b_gru_ref,
                 Wg_moeT_ref, bg_moe_ref, We1cat_ref, be1cat_ref,
                 We2stk_ref, be2_ref, blkones_ref,
                 g_moe_ref, b_moe_ref,
                 WkvT_ref, bkv_ref, WqT_ref, bq_ref, WoT_ref, bo_ref,
                 g_mha_ref, b_mha_ref,
                 out_ref,
                 hin_ref, fw_ref, bw_ref):
    bn = BN_MAGE
    x = xT_ref[...].reshape(T * bn, F_IN)
    h_in = _mm(x, WinT_ref[...]) + bin_ref[...]
    hin_ref[...] = h_in

    Wcf = Wcf_ref[...]
    Wcb = Wcb_ref[...]
    bcf = bcf_ref[...]
    bcb = bcb_ref[...]

    def gru_step(xt, h, Wc, bc):
        cat = jnp.concatenate([xt, h], axis=1)
        g = _mm(cat, Wc) + bc
        r = jax.nn.sigmoid(g[:, :D])
        z = jax.nn.sigmoid(g[:, D:2 * D])
        n = jnp.tanh(g[:, 2 * D:3 * D] + r * g[:, 3 * D:])
        return (1.0 - z) * n + z * h

    def body(t, carry):
        h_f, h_b = carry
        xf = hin_ref[pl.ds(t * bn, bn), :]
        h_f = gru_step(xf, h_f, Wcf, bcf)
        fw_ref[pl.ds(t * bn, bn), :] = h_f
        tb = (T - 1) - t
        xb = hin_ref[pl.ds(tb * bn, bn), :]
        h_b = gru_step(xb, h_b, Wcb, bcb)
        bw_ref[pl.ds(tb * bn, bn), :] = h_b
        return h_f, h_b

    h0 = jnp.zeros((bn, D), jnp.float32)
    jax.lax.fori_loop(0, T, body, (h0, h0))

    fw = fw_ref[...]
    bw = bw_ref[...]
    gate = jax.nn.sigmoid(_mm(fw, WgfT_ref[...]) + bgf_ref[...]
                          + _mm(bw, WgbT_ref[...]))
    zg = gate * fw + (1.0 - gate) * bw
    zg = _ln_rows(zg + h_in, g_gru_ref[...], b_gru_ref[...])

    logits = _mm(zg, Wg_moeT_ref[...]) + bg_moe_ref[...]
    best = jnp.max(logits, axis=1, keepdims=True)
    pe = jnp.exp(logits - best)
    w_top = 1.0 / jnp.sum(pe, axis=1, keepdims=True)
    eq = jnp.where(logits == best, 1.0, 0.0)
    ri = jax.lax.broadcasted_iota(jnp.int32, (E_EXP, E_EXP), 0)
    ci = jax.lax.broadcasted_iota(jnp.int32, (E_EXP, E_EXP), 1)
    lt = jnp.where(ri <= ci, 1.0, 0.0)
    cum = _mm(eq, lt)
    oh4 = jnp.where(cum == 1.0, eq, 0.0)
    h1 = _mm(zg, We1cat_ref[...]) + be1cat_ref[...]
    h1 = 0.5 * h1 * (1.0 + jax.lax.erf(h1 * jnp.float32(0.7071067811865476)))
    mask512 = _mm(oh4, blkones_ref[...])
    acc = _mm(h1 * mask512, We2stk_ref[...])
    be2_sel = _mm(oh4, be2_ref[...])
    mo = w_top * (acc + be2_sel)
    zm = _ln_rows(zg + mo, g_moe_ref[...], b_moe_ref[...])

    zm3 = zm.reshape(T, bn, D)
    z_last = zm3[T - 1]
    kv = (_mm(zm, WkvT_ref[...]) + bkv_ref[...]).reshape(T, bn, 2 * D)
    q = _mm(z_last, WqT_ref[...]) + bq_ref[...]
    scale = 1.0 / jnp.sqrt(jnp.float32(DH_ATTN))
    outs = []
    for h in range(H_ATTN):
        qh = q[:, h * DH_ATTN:(h + 1) * DH_ATTN]
        kh = kv[:, :, h * DH_ATTN:(h + 1) * DH_ATTN]
        vh = kv[:, :, D + h * DH_ATTN:D + (h + 1) * DH_ATTN]
        s = jnp.sum(kh * qh[None, :, :], axis=-1) * scale
        sm = jnp.max(s, axis=0, keepdims=True)
        es = jnp.exp(s - sm)
        attn = es / jnp.sum(es, axis=0, keepdims=True)
        outs.append(jnp.sum(attn[:, :, None] * vh, axis=0))
    at = _mm(jnp.concatenate(outs, axis=1), WoT_ref[...]) + bo_ref[...]
    out_ref[...] = _ln_rows(z_last + at, g_mha_ref[...], b_mha_ref[...])


def _mage_call(xT, wd):
    grid = N // BN_MAGE
    rep = lambda *s: pl.BlockSpec(s, lambda i: (0,) * len(s))
    return pl.pallas_call(
        _mage_kernel,
        grid=(grid,),
        in_specs=[
            pl.BlockSpec((T, BN_MAGE, F_IN), lambda i: (0, i, 0)),
            rep(F_IN, D), rep(1, D),
            rep(2 * D, 4 * D), rep(1, 4 * D),
            rep(2 * D, 4 * D), rep(1, 4 * D),
            rep(D, D), rep(1, D), rep(D, D), rep(1, D), rep(1, D),
            rep(D, E_EXP), rep(1, E_EXP),
            rep(D, E_EXP * D), rep(1, E_EXP * D),
            rep(E_EXP * D, D), rep(E_EXP, D), rep(E_EXP, E_EXP * D),
            rep(1, D), rep(1, D),
            rep(D, 2 * D), rep(1, 2 * D), rep(D, D), rep(1, D),
            rep(D, D), rep(1, D), rep(1, D), rep(1, D),
        ],
        out_specs=pl.BlockSpec((BN_MAGE, D), lambda i: (i, 0)),
        out_shape=jax.ShapeDtypeStruct((N, D), jnp.float32),
        scratch_shapes=[
            pltpu.VMEM((T * BN_MAGE, D), jnp.float32),
            pltpu.VMEM((T * BN_MAGE, D), jnp.float32),
            pltpu.VMEM((T * BN_MAGE, D), jnp.float32),
        ],
        compiler_params=pltpu.CompilerParams(
            dimension_semantics=("arbitrary",),
            vmem_limit_bytes=100 * 1024 * 1024),
    )(xT, *wd)


def _gat_kernel(h_ref, hT_ref, adjp_ref, adjn_ref,
                Wtp_ref, WtTp_ref, WuTp_ref, Wvp_ref, bgp_ref,
                WprTp_ref, bprp_ref,
                Wtn_ref, WtTn_ref, WuTn_ref, Wvn_ref, bgn_ref,
                WprTn_ref, bprn_ref,
                Ws1T_ref, bs1_ref, Ws2T_ref,
                out_ref,
                supp_ref, supTp_ref, supn_ref, supTn_ref):
    pid = pl.program_id(0)

    @pl.when(pid == 0)
    def _init():
        hh = h_ref[...]
        hhT = hT_ref[...]
        supp_ref[...] = _mm(hh, Wtp_ref[...])
        supn_ref[...] = _mm(hh, Wtn_ref[...])
        supTp_ref[...] = _mm(WtTp_ref[...], hhT)
        supTn_ref[...] = _mm(WtTn_ref[...], hhT)

    i0 = pid * BN_GAT
    h_blk = h_ref[pl.ds(i0, BN_GAT), :]

    def gat_side(adj, sup_ref, supT_ref, ones_col, WuT, Wv, bg, WprT, bpr):
        outs = []
        for hh in range(H_GAT):
            c0 = hh * O_GAT
            uT = _mm(WuT[hh], supT_ref[pl.ds(c0, O_GAT), :])
            v = _mm(sup_ref[pl.ds(i0, BN_GAT), c0:c0 + O_GAT],
                    Wv[hh])
            w = uT + v
            w = jnp.where(w >= 0, w, 0.2 * w)
            masked = w * adj
            e = jnp.exp(jnp.where(masked != 0, masked, -jnp.inf))
            sup_aug = jnp.concatenate(
                [sup_ref[:, c0:c0 + O_GAT], ones_col], axis=1)
            r = _mm(e, sup_aug)
            outs.append(r[:, :O_GAT] / r[:, O_GAT:])
        o = jnp.concatenate(outs, axis=1) + bg
        return o + _mm(h_blk, WprT) + bpr

    ones_col = jnp.ones((N, 1), jnp.float32)
    hp = gat_side(adjp_ref[...], supp_ref, supTp_ref, ones_col,
                  WuTp_ref[...], Wvp_ref[...], bgp_ref[...],
                  WprTp_ref[...], bprp_ref[...])
    hn = gat_side(adjn_ref[...], supn_ref, supTn_ref, ones_col,
                  WuTn_ref[...], Wvn_ref[...], bgn_ref[...],
                  WprTn_ref[...], bprn_ref[...])

    Ws1T = Ws1T_ref[...]
    bs1 = bs1_ref[...]
    Ws2T = Ws2T_ref[...]
    scores = [_mm(jnp.tanh(_mm(part, Ws1T) + bs1), Ws2T)
              for part in (h_blk, hp, hn)]
    sm = jnp.maximum(jnp.maximum(scores[0], scores[1]), scores[2])
    es = [jnp.exp(s - sm) for s in scores]
    tot = es[0] + es[1] + es[2]
    out_ref[...] = (es[0] * h_blk + es[1] * hp + es[2] * hn) / tot


def _gat_call(h, hT, pos_adj, neg_adj, wd):
    grid = N // BN_GAT
    rep = lambda *s: pl.BlockSpec(s, lambda i: (0,) * len(s))
    return pl.pallas_call(
        _gat_kernel,
        grid=(grid,),
        in_specs=[
            rep(N, D), rep(D, N),
            pl.BlockSpec((BN_GAT, N), lambda i: (i, 0)),
            pl.BlockSpec((BN_GAT, N), lambda i: (i, 0)),
            rep(D, D), rep(D, D), rep(H_GAT, 1, O_GAT), rep(H_GAT, O_GAT, 1),
            rep(1, D), rep(D, D), rep(1, D),
            rep(D, D), rep(D, D), rep(H_GAT, 1, O_GAT), rep(H_GAT, O_GAT, 1),
            rep(1, D), rep(D, D), rep(1, D),
            rep(D, D), rep(1, D), rep(D, 1),
        ],
        out_specs=pl.BlockSpec((BN_GAT, D), lambda i: (i, 0)),
        out_shape=jax.ShapeDtypeStruct((N, D), jnp.float32),
        scratch_shapes=[
            pltpu.VMEM((N, D), jnp.float32),
            pltpu.VMEM((D, N), jnp.float32),
            pltpu.VMEM((N, D), jnp.float32),
            pltpu.VMEM((D, N), jnp.float32),
        ],
        compiler_params=pltpu.CompilerParams(
            dimension_semantics=("arbitrary",)),
    )(h, hT, pos_adj, neg_adj, *wd)


def _gph_kernel(hs_ref, WthT_ref, bth_ref, WpgT_ref, g_ref, b_ref,
                WpredT_ref, bpred_ref, out_ref):
    eps = 1e-8
    hs = hs_ref[...]
    Hr = _mm(hs, WthT_ref[...]) + bth_ref[...]
    Hr = jnp.where(Hr <= 0, 0.0, jnp.tanh(Hr))
    cm = jnp.max(Hr, axis=0, keepdims=True)
    ce = jnp.exp(Hr - cm)
    Hg = ce / jnp.sum(ce, axis=0, keepdims=True)

    mu = jnp.zeros((1, M_HYPER), jnp.float32)
    for a in range(M_HYPER):
        p = Hg[:, a:a + 1]
        mm2 = 0.5 * (p + Hg)
        kl_pm = jnp.sum(
            p * jnp.log(jnp.clip(p / (mm2 + eps), eps, None)), axis=0,
            keepdims=True)
        kl_qm = jnp.sum(
            Hg * jnp.log(jnp.clip(Hg / (mm2 + eps), eps, None)), axis=0,
            keepdims=True)
        row = jnp.clip(0.5 * (kl_pm + kl_qm), 0.0, None)
        mu = mu + row / jnp.float32(M_HYPER)

    mu_mean = jnp.mean(mu)
    dev = mu - mu_mean
    std = jnp.sqrt(jnp.sum(dev * dev) / jnp.float32(M_HYPER - 1))
    mu_z = dev / (std + eps)
    wm = jnp.max(mu_z)
    we = jnp.exp(mu_z - wm)
    w = we / jnp.sum(we)

    HW = Hg * w
    hp2 = _mm(hs, WpgT_ref[...])
    tmp = jax.lax.dot_general(Hg, hp2, (((0,), (0,)), ((), ())),
                              preferred_element_type=jnp.float32)
    z = _mm(HW, tmp)
    z = jnp.where(z > 0, z, jnp.exp(jnp.minimum(z, 0.0)) - 1.0)
    zf = _ln_rows(z + hs, g_ref[...], b_ref[...])
    out_ref[...] = _mm(zf, WpredT_ref[...]) + bpred_ref[...]


def _gph_call(hs, wd):
    rep = lambda *s: pl.BlockSpec(s, lambda: (0,) * len(s))
    return pl.pallas_call(
        _gph_kernel,
        in_specs=[rep(N, D), rep(D, M_HYPER), rep(1, M_HYPER), rep(D, D),
                  rep(1, D), rep(1, D), rep(D, 1), rep(1, 1)],
        out_specs=rep(N, 1),
        out_shape=jax.ShapeDtypeStruct((N, 1), jnp.float32),
    )(hs, *wd)


def _gru_cat_weights(Wih, Whh, bih, bhh):
    z128 = jnp.zeros((D, D), jnp.float32)
    top = jnp.concatenate([Wih[:D].T, Wih[D:2 * D].T, Wih[2 * D:].T, z128],
                          axis=1)
    bot = jnp.concatenate([Whh[:D].T, Whh[D:2 * D].T, z128, Whh[2 * D:].T],
                          axis=1)
    Wc = jnp.concatenate([top, bot], axis=0)
    bc = jnp.concatenate([(bih[:D] + bhh[:D]),
                          (bih[D:2 * D] + bhh[D:2 * D]),
                          bih[2 * D:], bhh[2 * D:]])[None, :]
    return Wc, bc


def _row(v):
    return v.reshape(1, -1).astype(jnp.float32)


@jax.jit
def kernel(x, pos_adj, neg_adj, params):
    p = params
    xT = jnp.swapaxes(x, 0, 1)

    Wcf, bcf = _gru_cat_weights(p['Wih_f'], p['Whh_f'], p['bih_f'], p['bhh_f'])
    Wcb, bcb = _gru_cat_weights(p['Wih_b'], p['Whh_b'], p['bih_b'], p['bhh_b'])
    Wqkv = p['Wqkv']
    mage_w = (
        p['W_in'].T, _row(p['b_in']), Wcf, bcf, Wcb, bcb,
        p['Wgf'].T, _row(p['bgf']), p['Wgb'].T, _row(p['g_gru']),
        _row(p['b_gru']),
        p['Wg'].T, _row(p['bg']),
        jnp.concatenate([p['We1'][e].T for e in range(E_EXP)], axis=1),
        p['be1'].reshape(1, E_EXP * D),
        jnp.concatenate([p['We2'][e].T for e in range(E_EXP)], axis=0),
        p['be2'],
        jnp.kron(jnp.eye(E_EXP, dtype=jnp.float32),
                 jnp.ones((1, D), jnp.float32)),
        _row(p['g_moe']), _row(p['b_moe']),
        Wqkv[D:].T, _row(p['bqkv'][D:]), Wqkv[:D].T, _row(p['bqkv'][:D]),
        p['Wo'].T, _row(p['bo']), _row(p['g_mha']), _row(p['b_mha']),
    )
    h = _mage_call(xT, mage_w)

    gat_w = (
        p['Wt_p'], p['Wt_p'].T,
        jnp.swapaxes(p['Wu_p'], 1, 2), p['Wv_p'], p['bg_p'],
        p['Wpr_p'].T, _row(p['bpr_p']),
        p['Wt_n'], p['Wt_n'].T,
        jnp.swapaxes(p['Wu_n'], 1, 2), p['Wv_n'], p['bg_n'],
        p['Wpr_n'].T, _row(p['bpr_n']),
        p['Ws1'].T, _row(p['bs1']), p['Ws2'].T,
    )
    hs = _gat_call(h, h.T, pos_adj, neg_adj, gat_w)

    gph_w = (p['Wth'].T, _row(p['bth']), p['Wpg'].T,
             _row(p['g_gph']), _row(p['b_gph']),
             p['Wpred'].T, _row(p['bpred']))
    return _gph_call(hs, gph_w)


# speedup vs baseline: 1.1313x; 1.1313x over previous
import functools

import jax
import jax.numpy as jnp
from jax.experimental import pallas as pl
from jax.experimental.pallas import tpu as pltpu

N, T, F_IN, D = 2000, 32, 64, 128
M_HYPER, E_EXP, H_GAT, H_ATTN = 32, 4, 4, 2
O_GAT = D // H_GAT
DH_ATTN = D // H_ATTN

BN_MAGE = 200
BN_GAT = 400
LN_EPS = 1e-5


def _ln_rows(x, g, b):
    mu = x.mean(-1, keepdims=True)
    var = ((x - mu) ** 2).mean(-1, keepdims=True)
    return (x - mu) / jnp.sqrt(var + LN_EPS) * g + b


def _mm(a, b):
    return jax.lax.dot_general(a, b, (((1,), (0,)), ((), ())),
                               preferred_element_type=jnp.float32)


def _mage_kernel(xT_ref, WinT_ref, bin_ref, Wcf_ref, bcf_ref, Wcb_ref, bcb_ref,
                 WgfT_ref, bgf_ref, WgbT_ref, g_gru_ref, b_gru_ref,
                 Wg_moeT_ref, bg_moe_ref, We1cat_ref, be1cat_ref,
                 We2stk_ref, be2_ref,
                 g_moe_ref, b_moe_ref,
                 WkvT_ref, bkv_ref, WqT_ref, bq_ref, WoT_ref, bo_ref,
                 g_mha_ref, b_mha_ref,
                 out_ref,
                 hin_ref, fw_ref, bw_ref):
    bn = BN_MAGE
    x = xT_ref[...].reshape(T * bn, F_IN)
    h_in = _mm(x, WinT_ref[...]) + bin_ref[...]
    hin_ref[...] = h_in

    Wcf = Wcf_ref[...]
    Wcb = Wcb_ref[...]
    bcf = bcf_ref[...]
    bcb = bcb_ref[...]

    def gru_step(xt, h, Wc, bc):
        cat = jnp.concatenate([xt, h], axis=1)
        g = _mm(cat, Wc) + bc
        r = jax.nn.sigmoid(g[:, :D])
        z = jax.nn.sigmoid(g[:, D:2 * D])
        n = jnp.tanh(g[:, 2 * D:3 * D] + r * g[:, 3 * D:])
        return (1.0 - z) * n + z * h

    def body(t, carry):
        h_f, h_b = carry
        xf = hin_ref[pl.ds(t * bn, bn), :]
        h_f = gru_step(xf, h_f, Wcf, bcf)
        fw_ref[pl.ds(t * bn, bn), :] = h_f
        tb = (T - 1) - t
        xb = hin_ref[pl.ds(tb * bn, bn), :]
        h_b = gru_step(xb, h_b, Wcb, bcb)
        bw_ref[pl.ds(tb * bn, bn), :] = h_b
        return h_f, h_b

    h0 = jnp.zeros((bn, D), jnp.float32)
    jax.lax.fori_loop(0, T, body, (h0, h0))

    fw = fw_ref[...]
    bw = bw_ref[...]
    gate = jax.nn.sigmoid(_mm(fw, WgfT_ref[...]) + bgf_ref[...]
                          + _mm(bw, WgbT_ref[...]))
    zg = gate * fw + (1.0 - gate) * bw
    zg = _ln_rows(zg + h_in, g_gru_ref[...], b_gru_ref[...])

    logits = _mm(zg, Wg_moeT_ref[...]) + bg_moe_ref[...]
    best = logits[:, 0:1]
    choice = jnp.zeros((T * bn, 1), jnp.int32)
    for e in range(1, E_EXP):
        le = logits[:, e:e + 1]
        better = le > best
        best = jnp.where(better, le, best)
        choice = jnp.where(better, jnp.int32(e), choice)
    pe = jnp.exp(logits - best)
    w_top = 1.0 / jnp.sum(pe, axis=1, keepdims=True)
    h1 = _mm(zg, We1cat_ref[...]) + be1cat_ref[...]
    h1 = 0.5 * h1 * (1.0 + jax.lax.erf(h1 * jnp.float32(0.7071067811865476)))
    lane = jax.lax.broadcasted_iota(jnp.int32, (1, E_EXP * D), 1)
    colgrp = jax.lax.shift_right_logical(lane, 7)
    h1m = jnp.where(colgrp == choice, h1, 0.0)
    acc = _mm(h1m, We2stk_ref[...])
    lane4 = jax.lax.broadcasted_iota(jnp.int32, (1, E_EXP), 1)
    oh4 = jnp.where(lane4 == choice, 1.0, 0.0)
    be2_sel = _mm(oh4, be2_ref[...])
    mo = w_top * (acc + be2_sel)
    zm = _ln_rows(zg + mo, g_moe_ref[...], b_moe_ref[...])

    zm3 = zm.reshape(T, bn, D)
    z_last = zm3[T - 1]
    kv = (_mm(zm, WkvT_ref[...]) + bkv_ref[...]).reshape(T, bn, 2 * D)
    q = _mm(z_last, WqT_ref[...]) + bq_ref[...]
    scale = 1.0 / jnp.sqrt(jnp.float32(DH_ATTN))
    outs = []
    for h in range(H_ATTN):
        qh = q[:, h * DH_ATTN:(h + 1) * DH_ATTN]
        kh = kv[:, :, h * DH_ATTN:(h + 1) * DH_ATTN]
        vh = kv[:, :, D + h * DH_ATTN:D + (h + 1) * DH_ATTN]
        s = jnp.sum(kh * qh[None, :, :], axis=-1) * scale
        sm = jnp.max(s, axis=0, keepdims=True)
        es = jnp.exp(s - sm)
        attn = es / jnp.sum(es, axis=0, keepdims=True)
        outs.append(jnp.sum(attn[:, :, None] * vh, axis=0))
    at = _mm(jnp.concatenate(outs, axis=1), WoT_ref[...]) + bo_ref[...]
    out_ref[...] = _ln_rows(z_last + at, g_mha_ref[...], b_mha_ref[...])


def _mage_call(xT, wd):
    grid = N // BN_MAGE
    rep = lambda *s: pl.BlockSpec(s, lambda i: (0,) * len(s))
    return pl.pallas_call(
        _mage_kernel,
        grid=(grid,),
        in_specs=[
            pl.BlockSpec((T, BN_MAGE, F_IN), lambda i: (0, i, 0)),
            rep(F_IN, D), rep(1, D),
            rep(2 * D, 4 * D), rep(1, 4 * D),
            rep(2 * D, 4 * D), rep(1, 4 * D),
            rep(D, D), rep(1, D), rep(D, D), rep(1, D), rep(1, D),
            rep(D, E_EXP), rep(1, E_EXP),
            rep(D, E_EXP * D), rep(1, E_EXP * D),
            rep(E_EXP * D, D), rep(E_EXP, D),
            rep(1, D), rep(1, D),
            rep(D, 2 * D), rep(1, 2 * D), rep(D, D), rep(1, D),
            rep(D, D), rep(1, D), rep(1, D), rep(1, D),
        ],
        out_specs=pl.BlockSpec((BN_MAGE, D), lambda i: (i, 0)),
        out_shape=jax.ShapeDtypeStruct((N, D), jnp.float32),
        scratch_shapes=[
            pltpu.VMEM((T * BN_MAGE, D), jnp.float32),
            pltpu.VMEM((T * BN_MAGE, D), jnp.float32),
            pltpu.VMEM((T * BN_MAGE, D), jnp.float32),
        ],
        compiler_params=pltpu.CompilerParams(
            dimension_semantics=("arbitrary",),
            vmem_limit_bytes=100 * 1024 * 1024),
    )(xT, *wd)


def _gat_kernel(h_ref, hT_ref, adjp_ref, adjn_ref,
                Wtp_ref, WtTp_ref, WuTp_ref, Wvp_ref, bgp_ref,
                WprTp_ref, bprp_ref,
                Wtn_ref, WtTn_ref, WuTn_ref, Wvn_ref, bgn_ref,
                WprTn_ref, bprn_ref,
                Ws1T_ref, bs1_ref, Ws2T_ref,
                out_ref,
                supp_ref, supTp_ref, supn_ref, supTn_ref):
    pid = pl.program_id(0)

    @pl.when(pid == 0)
    def _init():
        hh = h_ref[...]
        hhT = hT_ref[...]
        supp_ref[...] = _mm(hh, Wtp_ref[...])
        supn_ref[...] = _mm(hh, Wtn_ref[...])
        supTp_ref[...] = _mm(WtTp_ref[...], hhT)
        supTn_ref[...] = _mm(WtTn_ref[...], hhT)

    i0 = pid * BN_GAT
    h_blk = h_ref[pl.ds(i0, BN_GAT), :]

    def gat_side(adj, sup_ref, supT_ref, ones_col, WuT, Wv, bg, WprT, bpr):
        outs = []
        for hh in range(H_GAT):
            c0 = hh * O_GAT
            uT = _mm(WuT[hh], supT_ref[pl.ds(c0, O_GAT), :])
            v = _mm(sup_ref[pl.ds(i0, BN_GAT), c0:c0 + O_GAT],
                    Wv[hh])
            w = uT + v
            w = jnp.where(w >= 0, w, 0.2 * w)
            masked = w * adj
            e = jnp.exp(jnp.where(masked != 0, masked, -jnp.inf))
            sup_aug = jnp.concatenate(
                [sup_ref[:, c0:c0 + O_GAT], ones_col], axis=1)
            r = _mm(e, sup_aug)
            outs.append(r[:, :O_GAT] / r[:, O_GAT:])
        o = jnp.concatenate(outs, axis=1) + bg
        return o + _mm(h_blk, WprT) + bpr

    ones_col = jnp.ones((N, 1), jnp.float32)
    hp = gat_side(adjp_ref[...], supp_ref, supTp_ref, ones_col,
                  WuTp_ref[...], Wvp_ref[...], bgp_ref[...],
                  WprTp_ref[...], bprp_ref[...])
    hn = gat_side(adjn_ref[...], supn_ref, supTn_ref, ones_col,
                  WuTn_ref[...], Wvn_ref[...], bgn_ref[...],
                  WprTn_ref[...], bprn_ref[...])

    Ws1T = Ws1T_ref[...]
    bs1 = bs1_ref[...]
    Ws2T = Ws2T_ref[...]
    scores = [_mm(jnp.tanh(_mm(part, Ws1T) + bs1), Ws2T)
              for part in (h_blk, hp, hn)]
    sm = jnp.maximum(jnp.maximum(scores[0], scores[1]), scores[2])
    es = [jnp.exp(s - sm) for s in scores]
    tot = es[0] + es[1] + es[2]
    out_ref[...] = (es[0] * h_blk + es[1] * hp + es[2] * hn) / tot


def _gat_call(h, hT, pos_adj, neg_adj, wd):
    grid = N // BN_GAT
    rep = lambda *s: pl.BlockSpec(s, lambda i: (0,) * len(s))
    return pl.pallas_call(
        _gat_kernel,
        grid=(grid,),
        in_specs=[
            rep(N, D), rep(D, N),
            pl.BlockSpec((BN_GAT, N), lambda i: (i, 0)),
            pl.BlockSpec((BN_GAT, N), lambda i: (i, 0)),
            rep(D, D), rep(D, D), rep(H_GAT, 1, O_GAT), rep(H_GAT, O_GAT, 1),
            rep(1, D), rep(D, D), rep(1, D),
            rep(D, D), rep(D, D), rep(H_GAT, 1, O_GAT), rep(H_GAT, O_GAT, 1),
            rep(1, D), rep(D, D), rep(1, D),
            rep(D, D), rep(1, D), rep(D, 1),
        ],
        out_specs=pl.BlockSpec((BN_GAT, D), lambda i: (i, 0)),
        out_shape=jax.ShapeDtypeStruct((N, D), jnp.float32),
        scratch_shapes=[
            pltpu.VMEM((N, D), jnp.float32),
            pltpu.VMEM((D, N), jnp.float32),
            pltpu.VMEM((N, D), jnp.float32),
            pltpu.VMEM((D, N), jnp.float32),
        ],
        compiler_params=pltpu.CompilerParams(
            dimension_semantics=("arbitrary",)),
    )(h, hT, pos_adj, neg_adj, *wd)


def _gph_kernel(hs_ref, WthT_ref, bth_ref, WpgT_ref, g_ref, b_ref,
                WpredT_ref, bpred_ref, out_ref):
    eps = 1e-8
    hs = hs_ref[...]
    Hr = _mm(hs, WthT_ref[...]) + bth_ref[...]
    Hr = jnp.where(Hr <= 0, 0.0, jnp.tanh(Hr))
    cm = jnp.max(Hr, axis=0, keepdims=True)
    ce = jnp.exp(Hr - cm)
    Hg = ce / jnp.sum(ce, axis=0, keepdims=True)

    Q = jnp.concatenate([Hg, Hg, Hg, Hg], axis=1)
    ri = jax.lax.broadcasted_iota(jnp.int32, (M_HYPER, 4 * M_HYPER), 0)
    ci = jax.lax.broadcasted_iota(jnp.int32, (M_HYPER, 4 * M_HYPER), 1)
    cgrp = jax.lax.shift_right_logical(ci, 5)
    sri = jax.lax.broadcasted_iota(jnp.int32, (4 * M_HYPER, M_HYPER), 0)
    sci = jax.lax.broadcasted_iota(jnp.int32, (4 * M_HYPER, M_HYPER), 1)
    sgrp = jax.lax.shift_right_logical(sri, 5)
    mu = jnp.zeros((1, M_HYPER), jnp.float32)
    for g in range(M_HYPER // 4):
        rp = jnp.where(ri == 4 * g + cgrp, 1.0, 0.0)
        P = _mm(Hg, rp)
        m2e = 0.5 * (P + Q) + eps
        tot = (P * jnp.log(jnp.clip(P / m2e, eps, None))
               + Q * jnp.log(jnp.clip(Q / m2e, eps, None)))
        row = jnp.sum(tot, axis=0, keepdims=True)
        rowc = jnp.clip(0.5 * row, 0.0, None)
        sel_g = jnp.where(sci == 4 * g + sgrp, 1.0, 0.0)
        mu = mu + _mm(rowc, sel_g) / jnp.float32(M_HYPER)


    mu_mean = jnp.mean(mu)
    dev = mu - mu_mean
    std = jnp.sqrt(jnp.sum(dev * dev) / jnp.float32(M_HYPER - 1))
    mu_z = dev / (std + eps)
    wm = jnp.max(mu_z)
    we = jnp.exp(mu_z - wm)
    w = we / jnp.sum(we)

    HW = Hg * w
    hp2 = _mm(hs, WpgT_ref[...])
    tmp = jax.lax.dot_general(Hg, hp2, (((0,), (0,)), ((), ())),
                              preferred_element_type=jnp.float32)
    z = _mm(HW, tmp)
    z = jnp.where(z > 0, z, jnp.exp(jnp.minimum(z, 0.0)) - 1.0)
    zf = _ln_rows(z + hs, g_ref[...], b_ref[...])
    out_ref[...] = _mm(zf, WpredT_ref[...]) + bpred_ref[...]


def _gph_call(hs, wd):
    rep = lambda *s: pl.BlockSpec(s, lambda: (0,) * len(s))
    return pl.pallas_call(
        _gph_kernel,
        in_specs=[rep(N, D), rep(D, M_HYPER), rep(1, M_HYPER), rep(D, D),
                  rep(1, D), rep(1, D), rep(D, 1), rep(1, 1)],
        out_specs=rep(N, 1),
        out_shape=jax.ShapeDtypeStruct((N, 1), jnp.float32),
    )(hs, *wd)


def _gru_cat_weights(Wih, Whh, bih, bhh):
    z128 = jnp.zeros((D, D), jnp.float32)
    top = jnp.concatenate([Wih[:D].T, Wih[D:2 * D].T, Wih[2 * D:].T, z128],
                          axis=1)
    bot = jnp.concatenate([Whh[:D].T, Whh[D:2 * D].T, z128, Whh[2 * D:].T],
                          axis=1)
    Wc = jnp.concatenate([top, bot], axis=0)
    bc = jnp.concatenate([(bih[:D] + bhh[:D]),
                          (bih[D:2 * D] + bhh[D:2 * D]),
                          bih[2 * D:], bhh[2 * D:]])[None, :]
    return Wc, bc


def _row(v):
    return v.reshape(1, -1).astype(jnp.float32)


@jax.jit
def kernel(x, pos_adj, neg_adj, params):
    p = params
    xT = jnp.swapaxes(x, 0, 1)

    Wcf, bcf = _gru_cat_weights(p['Wih_f'], p['Whh_f'], p['bih_f'], p['bhh_f'])
    Wcb, bcb = _gru_cat_weights(p['Wih_b'], p['Whh_b'], p['bih_b'], p['bhh_b'])
    Wqkv = p['Wqkv']
    mage_w = (
        p['W_in'].T, _row(p['b_in']), Wcf, bcf, Wcb, bcb,
        p['Wgf'].T, _row(p['bgf']), p['Wgb'].T, _row(p['g_gru']),
        _row(p['b_gru']),
        p['Wg'].T, _row(p['bg']),
        jnp.concatenate([p['We1'][e].T for e in range(E_EXP)], axis=1),
        p['be1'].reshape(1, E_EXP * D),
        jnp.concatenate([p['We2'][e].T for e in range(E_EXP)], axis=0),
        p['be2'],
        _row(p['g_moe']), _row(p['b_moe']),
        Wqkv[D:].T, _row(p['bqkv'][D:]), Wqkv[:D].T, _row(p['bqkv'][:D]),
        p['Wo'].T, _row(p['bo']), _row(p['g_mha']), _row(p['b_mha']),
    )
    h = _mage_call(xT, mage_w)

    gat_w = (
        p['Wt_p'], p['Wt_p'].T,
        jnp.swapaxes(p['Wu_p'], 1, 2), p['Wv_p'], p['bg_p'],
        p['Wpr_p'].T, _row(p['bpr_p']),
        p['Wt_n'], p['Wt_n'].T,
        jnp.swapaxes(p['Wu_n'], 1, 2), p['Wv_n'], p['bg_n'],
        p['Wpr_n'].T, _row(p['bpr_n']),
        p['Ws1'].T, _row(p['bs1']), p['Ws2'].T,
    )
    hs = _gat_call(h, h.T, pos_adj, neg_adj, gat_w)

    gph_w = (p['Wth'].T, _row(p['bth']), p['Wpg'].T,
             _row(p['g_gph']), _row(p['b_gph']),
             p['Wpred'].T, _row(p['bpred']))
    return _gph_call(hs, gph_w)
